# baseline (speedup 1.0000x reference)
import functools
import math

import jax
import jax.numpy as jnp
from jax import lax
from jax.experimental import pallas as pl
from jax.experimental.pallas import tpu as pltpu
from jax.experimental.pallas import tpu_sc as plsc

D_MODEL = 64
SCALE = math.sqrt(D_MODEL)
LANES = 16
PADW = 128

NUM_CORES = 2
NUM_SUBCORES = 16
NW = NUM_CORES * NUM_SUBCORES

BBLK = 128


def _mesh():
    return plsc.VectorSubcoreMesh(core_axis_name="c", subcore_axis_name="s")


def _params():
    return pltpu.CompilerParams(use_tc_tiling_on_sc=True, needs_layout_passes=False)


@functools.partial(jax.jit, static_argnums=(2,))
def _pack_table(lutT, tail, V):
    BLKC = 2 * BBLK
    nfull = V // BLKC
    per_w = nfull // NW
    extra = nfull - per_w * NW

    @functools.partial(
        pl.kernel,
        out_type=jax.ShapeDtypeStruct((V // 2, PADW), jnp.float32),
        mesh=_mesh(),
        scratch_types=(
            [pltpu.VMEM((D_MODEL, BLKC), jnp.float32)] * 2
            + [pltpu.VMEM((BBLK, PADW), jnp.float32)] * 2
            + [pltpu.VMEM((D_MODEL, PADW), jnp.float32)]
            + [pltpu.SemaphoreType.DMA] * 4
        ),
        compiler_params=_params(),
    )
    def pack_kernel(lutT_hbm, tail_hbm, out_hbm,
                    in_0, in_1, out_0, out_1, tailv, is_0, is_1, os_0, os_1):
        ins = [in_0, in_1]
        outs = [out_0, out_1]
        isems = [is_0, is_1]
        osems = [os_0, os_1]
        w = lax.axis_index("s") * NUM_CORES + lax.axis_index("c")
        nb = per_w + jnp.where(w < extra, 1, 0)

        iot = lax.iota(jnp.int32, LANES)
        rots = [jnp.bitwise_and(iot + k, LANES - 1) for k in range(LANES)]

        def bid(t):
            return w + t * NW

        def istart(t, in_ref, sem):
            pltpu.async_copy(
                lutT_hbm.at[:, pl.ds(bid(t) * BLKC, BLKC)], in_ref, sem)

        def iwait(in_ref, sem):
            pltpu.make_async_copy(
                lutT_hbm.at[:, pl.ds(0, BLKC)], in_ref, sem).wait()

        def ostart(t, out_ref, sem):
            pltpu.async_copy(
                out_ref, out_hbm.at[pl.ds(bid(t) * BBLK, BBLK)], sem)

        def owait(out_ref, sem):
            pltpu.make_async_copy(
                out_ref, out_hbm.at[pl.ds(0, BBLK)], sem).wait()

        @pl.when(w == NW - 1)
        def _():
            pltpu.sync_copy(tail_hbm, tailv)

            def tq(q, c):
                for cg in range(PADW // LANES):
                    v = tailv[2 * q + (cg // 4), pl.ds((cg * LANES) % D_MODEL, LANES)]
                    out_0[q, pl.ds(cg * LANES, LANES)] = v
                return c

            lax.fori_loop(0, BBLK // 4, tq, 0)
            pltpu.sync_copy(out_0.at[pl.ds(0, BBLK // 4)],
                            out_hbm.at[pl.ds(V // 2 - BBLK // 4, BBLK // 4)])

        def shuffle(in_ref, out_ref):
            def jblock(jg, c):
                jloc = iot + jnp.bitwise_and(jg, 7) * LANES
                h = lax.shift_right_logical(jg, 3)
                col = jloc + h * BBLK
                dst0 = lax.shift_right_logical(jloc, 1) + h * (BBLK // 2)
                par = lax.shift_left(jnp.bitwise_and(jloc, 1), 6)

                def dblock(db, c2):
                    d0 = db * LANES
                    rowd = [rots[k] + d0 for k in range(LANES)]
                    vs = [plsc.load_gather(in_ref, [rowd[k], col])
                          for k in range(LANES)]
                    for k in range(LANES):
                        plsc.store_scatter(out_ref, [dst0, par + rowd[k]], vs[k])
                    return c2

                lax.fori_loop(0, D_MODEL // LANES, dblock, 0)
                return c

            lax.fori_loop(0, BLKC // LANES, jblock, 0)

        for i in range(2):
            @pl.when(nb > i)
            def _(i=i):
                istart(i, ins[i], isems[i])

        def step(tt, i):
            @pl.when(tt < nb)
            def _():
                iwait(ins[i], isems[i])

                @pl.when(tt >= 2)
                def _():
                    owait(outs[i], osems[i])

                shuffle(ins[i], outs[i])
                ostart(tt, outs[i], osems[i])

                @pl.when(tt + 2 < nb)
                def _():
                    istart(tt + 2, ins[i], isems[i])

        def tbody(t, c):
            for i in range(2):
                step(2 * t + i, i)
            return c

        lax.fori_loop(0, (per_w + 2) // 2, tbody, 0)

        for i in range(2):
            @pl.when(nb > i)
            def _(i=i):
                owait(outs[i], osems[i])

    return pack_kernel(lutT, tail)


@functools.partial(jax.jit, static_argnums=(2, 3))
def _embed(xT, lut2, S, B):
    @functools.partial(
        pl.kernel,
        out_type=jax.ShapeDtypeStruct((S, D_MODEL, B), jnp.float32),
        mesh=_mesh(),
        scratch_types=(
            [pltpu.VMEM((S, BBLK), jnp.int32)]
            + [pltpu.VMEM((BBLK,), jnp.int32)] * 3
            + [pltpu.VMEM((BBLK, PADW), jnp.float32)] * 3
            + [pltpu.VMEM((D_MODEL, BBLK), jnp.float32)] * 3
            + [pltpu.SemaphoreType.DMA] * 6
        ),
        compiler_params=_params(),
    )
    def emb_kernel(x_hbm, lut_hbm, out_hbm, idx_all, pair_0, pair_1, pair_2,
                   rows_0, rows_1, rows_2, out_0, out_1, out_2,
                   gs_0, gs_1, gs_2, os_0, os_1, os_2):
        pairs = [pair_0, pair_1, pair_2]
        rows = [rows_0, rows_1, rows_2]
        outs = [out_0, out_1, out_2]
        gsems = [gs_0, gs_1, gs_2]
        osems = [os_0, os_1, os_2]
        w = lax.axis_index("s") * NUM_CORES + lax.axis_index("c")
        b0 = w * BBLK
        pltpu.sync_copy(x_hbm.at[:, pl.ds(b0, BBLK)], idx_all)

        iot = lax.iota(jnp.int32, LANES)
        rots = [jnp.bitwise_and(iot + k, LANES - 1) for k in range(LANES)]

        def gstart(s, pair_ref, rows_ref, sem):
            for jg in range(BBLK // LANES):
                sl = pl.ds(jg * LANES, LANES)
                pair_ref[sl] = lax.shift_right_logical(idx_all[s, sl], 1)
            pltpu.async_copy(lut_hbm.at[pair_ref], rows_ref, sem)

        def gwait(pair_ref, rows_ref, sem):
            pltpu.make_async_copy(lut_hbm.at[pair_ref], rows_ref, sem).wait()

        def ostart(s, out_ref, sem):
            pltpu.async_copy(out_ref, out_hbm.at[s, :, pl.ds(b0, BBLK)], sem)

        def owait(out_ref, sem):
            pltpu.make_async_copy(
                out_ref, out_hbm.at[0, :, pl.ds(b0, BBLK)], sem).wait()

        def transpose_scale(s, rows_ref, out_ref):
            halfs = []
            rowvs = []
            for jg in range(BBLK // LANES):
                sl = pl.ds(jg * LANES, LANES)
                halfs.append(lax.shift_left(jnp.bitwise_and(idx_all[s, sl], 1), 6))
                rowvs.append(iot + (jg * LANES))

            def dblock(db, c):
                d0 = db * LANES
                rowd = [rots[k] + d0 for k in range(LANES)]
                for jg in range(BBLK // LANES):
                    base = halfs[jg] + d0
                    vs = [plsc.load_gather(rows_ref, [rowvs[jg], base + rots[k]])
                          for k in range(LANES)]
                    for k in range(LANES):
                        plsc.store_scatter(out_ref, [rowd[k], rowvs[jg]],
                                           vs[k] * SCALE)
                return c

            lax.fori_loop(0, D_MODEL // LANES, dblock, 0)

        for i in range(3):
            gstart(i, pairs[i], rows[i], gsems[i])

        def step(s, i):
            @pl.when(s < S)
            def _():
                gwait(pairs[i], rows[i], gsems[i])

                @pl.when(s >= 3)
                def _():
                    owait(outs[i], osems[i])

                transpose_scale(s, rows[i], outs[i])
                ostart(s, outs[i], osems[i])

                @pl.when(s + 3 < S)
                def _():
                    gstart(s + 3, pairs[i], rows[i], gsems[i])

        def kbody(k, c):
            for i in range(3):
                step(3 * k + i, i)
            return c

        lax.fori_loop(0, (S + 2) // 3, kbody, 0)
        for i in range(3):
            owait(outs[i], osems[i])

    return emb_kernel(xT, lut2)


def kernel(x, lut):
    b, s = x.shape
    v, d = lut.shape
    xT = jnp.swapaxes(x, 0, 1).astype(jnp.int32)
    lutT = jnp.swapaxes(lut, 0, 1)
    ntail = v % BBLK
    tail = jnp.pad(lut[v - ntail:, :], ((0, 0), (0, PADW - d)))
    lut2 = _pack_table(lutT, tail, v)
    out = _embed(xT, lut2, s, b)
    return jnp.transpose(out, (2, 0, 1))


# speedup vs baseline: 1.0147x; 1.0147x over previous
import functools
import math

import jax
import jax.numpy as jnp
from jax import lax
from jax.experimental import pallas as pl
from jax.experimental.pallas import tpu as pltpu
from jax.experimental.pallas import tpu_sc as plsc

D_MODEL = 64
SCALE = math.sqrt(D_MODEL)
LANES = 16
PADW = 128

NUM_CORES = 2
NUM_SUBCORES = 16
NW = NUM_CORES * NUM_SUBCORES

BBLK = 128


def _mesh():
    return plsc.VectorSubcoreMesh(core_axis_name="c", subcore_axis_name="s")


def _params():
    return pltpu.CompilerParams(use_tc_tiling_on_sc=True, needs_layout_passes=False)


@functools.partial(jax.jit, static_argnums=(2,))
def _pack_table(lutT, tail, V):
    nfull = V // BBLK
    per_w = nfull // NW
    extra = nfull - per_w * NW

    @functools.partial(
        pl.kernel,
        out_type=jax.ShapeDtypeStruct((V // 2, PADW), jnp.float32),
        mesh=_mesh(),
        scratch_types=(
            [pltpu.VMEM((D_MODEL, BBLK), jnp.float32)] * 4
            + [pltpu.VMEM((D_MODEL, PADW), jnp.float32)] * 4
            + [pltpu.SemaphoreType.DMA] * 8
        ),
        compiler_params=_params(),
    )
    def pack_kernel(lutT_hbm, tail_hbm, out_hbm,
                    in_0, in_1, in_2, in_3, out_0, out_1, out_2, out_3,
                    is_0, is_1, is_2, is_3, os_0, os_1, os_2, os_3):
        ins = [in_0, in_1, in_2, in_3]
        outs = [out_0, out_1, out_2, out_3]
        isems = [is_0, is_1, is_2, is_3]
        osems = [os_0, os_1, os_2, os_3]
        w = lax.axis_index("s") * NUM_CORES + lax.axis_index("c")
        nb = per_w + jnp.where(w < extra, 1, 0)

        iot = lax.iota(jnp.int32, LANES)
        rots = [jnp.bitwise_and(iot + k, LANES - 1) for k in range(LANES)]

        def bid(t):
            return w + t * NW

        def istart(t, in_ref, sem):
            pltpu.async_copy(
                lutT_hbm.at[:, pl.ds(bid(t) * BBLK, BBLK)], in_ref, sem)

        def iwait(in_ref, sem):
            pltpu.make_async_copy(
                lutT_hbm.at[:, pl.ds(0, BBLK)], in_ref, sem).wait()

        def ostart(t, out_ref, sem):
            pltpu.async_copy(
                out_ref, out_hbm.at[pl.ds(bid(t) * (BBLK // 2), D_MODEL)], sem)

        def owait(out_ref, sem):
            pltpu.make_async_copy(
                out_ref, out_hbm.at[pl.ds(0, D_MODEL)], sem).wait()

        @pl.when(w == NW - 1)
        def _():
            pltpu.sync_copy(tail_hbm, out_1)

            def tq(q, c):
                for cg in range(PADW // LANES):
                    v = out_1[2 * q + (cg // 4), pl.ds((cg * LANES) % D_MODEL, LANES)]
                    out_0[q, pl.ds(cg * LANES, LANES)] = v
                return c

            lax.fori_loop(0, BBLK // 4, tq, 0)
            pltpu.sync_copy(out_0.at[pl.ds(0, BBLK // 4)],
                            out_hbm.at[pl.ds(V // 2 - BBLK // 4, BBLK // 4)])

        def shuffle(in_ref, out_ref):
            for jg in range(BBLK // LANES):
                col = iot + (jg * LANES)
                dst0 = lax.shift_right_logical(col, 1)
                par = lax.shift_left(jnp.bitwise_and(col, 1), 6)

                def dblock(db, c):
                    d0 = db * LANES
                    rowd = [rots[k] + d0 for k in range(LANES)]
                    vs = [plsc.load_gather(in_ref, [rowd[k], col])
                          for k in range(LANES)]
                    for k in range(LANES):
                        plsc.store_scatter(out_ref, [dst0, par + rowd[k]], vs[k])
                    return c

                lax.fori_loop(0, D_MODEL // LANES, dblock, 0)

        for i in range(4):
            @pl.when(nb > i)
            def _(i=i):
                istart(i, ins[i], isems[i])

        def step(tt, i):
            @pl.when(tt < nb)
            def _():
                iwait(ins[i], isems[i])

                @pl.when(tt >= 4)
                def _():
                    owait(outs[i], osems[i])

                shuffle(ins[i], outs[i])
                ostart(tt, outs[i], osems[i])

                @pl.when(tt + 4 < nb)
                def _():
                    istart(tt + 4, ins[i], isems[i])

        def tbody(t, c):
            for i in range(4):
                step(4 * t + i, i)
            return c

        lax.fori_loop(0, (per_w + 4) // 4, tbody, 0)

        for i in range(4):
            @pl.when(nb > i)
            def _(i=i):
                owait(outs[i], osems[i])

    return pack_kernel(lutT, tail)


@functools.partial(jax.jit, static_argnums=(2, 3))
def _embed(xT, lut2, S, B):
    @functools.partial(
        pl.kernel,
        out_type=jax.ShapeDtypeStruct((S, D_MODEL, B), jnp.float32),
        mesh=_mesh(),
        scratch_types=(
            [pltpu.VMEM((S, BBLK), jnp.int32)]
            + [pltpu.VMEM((BBLK,), jnp.int32)] * 3
            + [pltpu.VMEM((BBLK, PADW), jnp.float32)] * 3
            + [pltpu.VMEM((D_MODEL, BBLK), jnp.float32)] * 3
            + [pltpu.SemaphoreType.DMA] * 6
        ),
        compiler_params=_params(),
    )
    def emb_kernel(x_hbm, lut_hbm, out_hbm, idx_all, pair_0, pair_1, pair_2,
                   rows_0, rows_1, rows_2, out_0, out_1, out_2,
                   gs_0, gs_1, gs_2, os_0, os_1, os_2):
        pairs = [pair_0, pair_1, pair_2]
        rows = [rows_0, rows_1, rows_2]
        outs = [out_0, out_1, out_2]
        gsems = [gs_0, gs_1, gs_2]
        osems = [os_0, os_1, os_2]
        w = lax.axis_index("s") * NUM_CORES + lax.axis_index("c")
        b0 = w * BBLK
        pltpu.sync_copy(x_hbm.at[:, pl.ds(b0, BBLK)], idx_all)

        iot = lax.iota(jnp.int32, LANES)
        rots = [jnp.bitwise_and(iot + k, LANES - 1) for k in range(LANES)]

        def gstart(s, pair_ref, rows_ref, sem):
            for jg in range(BBLK // LANES):
                sl = pl.ds(jg * LANES, LANES)
                pair_ref[sl] = lax.shift_right_logical(idx_all[s, sl], 1)
            pltpu.async_copy(lut_hbm.at[pair_ref], rows_ref, sem)

        def gwait(pair_ref, rows_ref, sem):
            pltpu.make_async_copy(lut_hbm.at[pair_ref], rows_ref, sem).wait()

        def ostart(s, out_ref, sem):
            pltpu.async_copy(out_ref, out_hbm.at[s, :, pl.ds(b0, BBLK)], sem)

        def owait(out_ref, sem):
            pltpu.make_async_copy(
                out_ref, out_hbm.at[0, :, pl.ds(b0, BBLK)], sem).wait()

        def transpose_scale(s, rows_ref, out_ref):
            halfs = []
            rowvs = []
            for jg in range(BBLK // LANES):
                sl = pl.ds(jg * LANES, LANES)
                halfs.append(lax.shift_left(jnp.bitwise_and(idx_all[s, sl], 1), 6))
                rowvs.append(iot + (jg * LANES))

            def dblock(db, c):
                d0 = db * LANES
                rowd = [rots[k] + d0 for k in range(LANES)]
                for jg in range(BBLK // LANES):
                    base = halfs[jg] + d0
                    vs = [plsc.load_gather(rows_ref, [rowvs[jg], base + rots[k]])
                          for k in range(LANES)]
                    for k in range(LANES):
                        plsc.store_scatter(out_ref, [rowd[k], rowvs[jg]],
                                           vs[k] * SCALE)
                return c

            lax.fori_loop(0, D_MODEL // LANES, dblock, 0)

        for i in range(3):
            gstart(i, pairs[i], rows[i], gsems[i])

        def step(s, i):
            @pl.when(s < S)
            def _():
                gwait(pairs[i], rows[i], gsems[i])

                @pl.when(s >= 3)
                def _():
                    owait(outs[i], osems[i])

                transpose_scale(s, rows[i], outs[i])
                ostart(s, outs[i], osems[i])

                @pl.when(s + 3 < S)
                def _():
                    gstart(s + 3, pairs[i], rows[i], gsems[i])

        def kbody(k, c):
            for i in range(3):
                step(3 * k + i, i)
            return c

        lax.fori_loop(0, (S + 2) // 3, kbody, 0)
        for i in range(3):
            owait(outs[i], osems[i])

    return emb_kernel(xT, lut2)


def kernel(x, lut):
    b, s = x.shape
    v, d = lut.shape
    xT = jnp.swapaxes(x, 0, 1).astype(jnp.int32)
    lutT = jnp.swapaxes(lut, 0, 1)
    ntail = v % BBLK
    tail = jnp.pad(lut[v - ntail:, :], ((0, 0), (0, PADW - d)))
    lut2 = _pack_table(lutT, tail, v)
    out = _embed(xT, lut2, s, b)
    return jnp.transpose(out, (2, 0, 1))


# speedup vs baseline: 1.0539x; 1.0386x over previous
import functools
import math

import jax
import jax.numpy as jnp
from jax import lax
from jax.experimental import pallas as pl
from jax.experimental.pallas import tpu as pltpu
from jax.experimental.pallas import tpu_sc as plsc

D_MODEL = 64
SCALE = math.sqrt(D_MODEL)
LANES = 16
PADW = 128

NUM_CORES = 2
NUM_SUBCORES = 16
NW = NUM_CORES * NUM_SUBCORES

BBLK = 128


def _mesh():
    return plsc.VectorSubcoreMesh(core_axis_name="c", subcore_axis_name="s")


def _params():
    return pltpu.CompilerParams(use_tc_tiling_on_sc=True, needs_layout_passes=False)


@functools.partial(jax.jit, static_argnums=(2,))
def _pack_table(lutT, tail, V):
    BLKC = 2 * BBLK
    nfull = V // BLKC
    per_w = nfull // NW
    extra = nfull - per_w * NW

    @functools.partial(
        pl.kernel,
        out_type=jax.ShapeDtypeStruct((V // 2, PADW), jnp.float32),
        mesh=_mesh(),
        scratch_types=(
            [pltpu.VMEM((D_MODEL, BLKC), jnp.float32)] * 3
            + [pltpu.VMEM((BBLK, PADW), jnp.float32)] * 2
            + [pltpu.SemaphoreType.DMA] * 5
        ),
        compiler_params=_params(),
    )
    def pack_kernel(lutT_hbm, tail_hbm, out_hbm,
                    in_0, in_1, in_2, out_0, out_1,
                    is_0, is_1, is_2, os_0, os_1):
        ins = [in_0, in_1, in_2]
        outs = [out_0, out_1]
        isems = [is_0, is_1, is_2]
        osems = [os_0, os_1]
        w = lax.axis_index("s") * NUM_CORES + lax.axis_index("c")
        nb = per_w + jnp.where(w < extra, 1, 0)

        iot = lax.iota(jnp.int32, LANES)
        rots = [jnp.bitwise_and(iot + k, LANES - 1) for k in range(LANES)]

        def bid(t):
            return w + t * NW

        def istart(t, in_ref, sem):
            pltpu.async_copy(
                lutT_hbm.at[:, pl.ds(bid(t) * BLKC, BLKC)], in_ref, sem)

        def iwait(in_ref, sem):
            pltpu.make_async_copy(
                lutT_hbm.at[:, pl.ds(0, BLKC)], in_ref, sem).wait()

        def ostart(t, out_ref, sem):
            pltpu.async_copy(
                out_ref, out_hbm.at[pl.ds(bid(t) * BBLK, BBLK)], sem)

        def owait(out_ref, sem):
            pltpu.make_async_copy(
                out_ref, out_hbm.at[pl.ds(0, BBLK)], sem).wait()

        @pl.when(w == NW - 1)
        def _():
            pltpu.sync_copy(tail_hbm, out_1.at[pl.ds(0, D_MODEL)])

            def tq(q, c):
                for cg in range(PADW // LANES):
                    v = out_1[2 * q + (cg // 4), pl.ds((cg * LANES) % D_MODEL, LANES)]
                    out_0[q, pl.ds(cg * LANES, LANES)] = v
                return c

            lax.fori_loop(0, BBLK // 4, tq, 0)
            pltpu.sync_copy(out_0.at[pl.ds(0, BBLK // 4)],
                            out_hbm.at[pl.ds(V // 2 - BBLK // 4, BBLK // 4)])

        jlocs = [iot + (jg * LANES) for jg in range(BBLK // LANES)]
        pars = [lax.shift_left(jnp.bitwise_and(jlocs[jg], 1), 6)
                for jg in range(BBLK // LANES)]

        def shuffle(in_ref, out_ref):
            def hblock(h, c):
                hcol = h * BBLK
                hrow = h * (BBLK // 2)
                for jg in range(BBLK // LANES):
                    col = jlocs[jg] + hcol
                    dst0 = lax.shift_right_logical(jlocs[jg], 1) + hrow
                    par = pars[jg]

                    def dblock(db, c2):
                        d0 = db * LANES
                        rowd = [rots[k] + d0 for k in range(LANES)]
                        vs = [plsc.load_gather(in_ref, [rowd[k], col])
                              for k in range(LANES)]
                        for k in range(LANES):
                            plsc.store_scatter(
                                out_ref, [dst0, par + rowd[k]], vs[k])
                        return c2

                    lax.fori_loop(0, D_MODEL // LANES, dblock, 0)
                return c

            lax.fori_loop(0, BLKC // BBLK, hblock, 0)

        for i in range(3):
            @pl.when(nb > i)
            def _(i=i):
                istart(i, ins[i], isems[i])

        def step(tt, i3, i2):
            @pl.when(tt < nb)
            def _():
                iwait(ins[i3], isems[i3])

                @pl.when(tt >= 2)
                def _():
                    owait(outs[i2], osems[i2])

                shuffle(ins[i3], outs[i2])
                ostart(tt, outs[i2], osems[i2])

                @pl.when(tt + 3 < nb)
                def _():
                    istart(tt + 3, ins[i3], isems[i3])

        def tbody(t, c):
            for i in range(6):
                step(6 * t + i, i % 3, i % 2)
            return c

        lax.fori_loop(0, (per_w + 6) // 6, tbody, 0)

        for i in range(2):
            @pl.when(nb > i)
            def _(i=i):
                owait(outs[i], osems[i])

    return pack_kernel(lutT, tail)


@functools.partial(jax.jit, static_argnums=(2, 3))
def _embed(xT, lut2, S, B):
    @functools.partial(
        pl.kernel,
        out_type=jax.ShapeDtypeStruct((S, D_MODEL, B), jnp.float32),
        mesh=_mesh(),
        scratch_types=(
            [pltpu.VMEM((S, BBLK), jnp.int32)]
            + [pltpu.VMEM((BBLK,), jnp.int32)] * 3
            + [pltpu.VMEM((BBLK, PADW), jnp.float32)] * 3
            + [pltpu.VMEM((D_MODEL, BBLK), jnp.float32)] * 3
            + [pltpu.SemaphoreType.DMA] * 6
        ),
        compiler_params=_params(),
    )
    def emb_kernel(x_hbm, lut_hbm, out_hbm, idx_all, pair_0, pair_1, pair_2,
                   rows_0, rows_1, rows_2, out_0, out_1, out_2,
                   gs_0, gs_1, gs_2, os_0, os_1, os_2):
        pairs = [pair_0, pair_1, pair_2]
        rows = [rows_0, rows_1, rows_2]
        outs = [out_0, out_1, out_2]
        gsems = [gs_0, gs_1, gs_2]
        osems = [os_0, os_1, os_2]
        w = lax.axis_index("s") * NUM_CORES + lax.axis_index("c")
        b0 = w * BBLK
        pltpu.sync_copy(x_hbm.at[:, pl.ds(b0, BBLK)], idx_all)

        iot = lax.iota(jnp.int32, LANES)
        rots = [jnp.bitwise_and(iot + k, LANES - 1) for k in range(LANES)]

        def gstart(s, pair_ref, rows_ref, sem):
            for jg in range(BBLK // LANES):
                sl = pl.ds(jg * LANES, LANES)
                pair_ref[sl] = lax.shift_right_logical(idx_all[s, sl], 1)
            pltpu.async_copy(lut_hbm.at[pair_ref], rows_ref, sem)

        def gwait(pair_ref, rows_ref, sem):
            pltpu.make_async_copy(lut_hbm.at[pair_ref], rows_ref, sem).wait()

        def ostart(s, out_ref, sem):
            pltpu.async_copy(out_ref, out_hbm.at[s, :, pl.ds(b0, BBLK)], sem)

        def owait(out_ref, sem):
            pltpu.make_async_copy(
                out_ref, out_hbm.at[0, :, pl.ds(b0, BBLK)], sem).wait()

        def transpose_scale(s, rows_ref, out_ref):
            halfs = []
            rowvs = []
            for jg in range(BBLK // LANES):
                sl = pl.ds(jg * LANES, LANES)
                halfs.append(lax.shift_left(jnp.bitwise_and(idx_all[s, sl], 1), 6))
                rowvs.append(iot + (jg * LANES))

            def dblock(db, c):
                d0 = db * LANES
                rowd = [rots[k] + d0 for k in range(LANES)]
                for jg in range(BBLK // LANES):
                    base = halfs[jg] + d0
                    vs = [plsc.load_gather(rows_ref, [rowvs[jg], base + rots[k]])
                          for k in range(LANES)]
                    for k in range(LANES):
                        plsc.store_scatter(out_ref, [rowd[k], rowvs[jg]],
                                           vs[k] * SCALE)
                return c

            lax.fori_loop(0, D_MODEL // LANES, dblock, 0)

        for i in range(3):
            gstart(i, pairs[i], rows[i], gsems[i])

        def step(s, i):
            @pl.when(s < S)
            def _():
                gwait(pairs[i], rows[i], gsems[i])

                @pl.when(s >= 3)
                def _():
                    owait(outs[i], osems[i])

                transpose_scale(s, rows[i], outs[i])
                ostart(s, outs[i], osems[i])

                @pl.when(s + 3 < S)
                def _():
                    gstart(s + 3, pairs[i], rows[i], gsems[i])

        def kbody(k, c):
            for i in range(3):
                step(3 * k + i, i)
            return c

        lax.fori_loop(0, (S + 2) // 3, kbody, 0)
        for i in range(3):
            owait(outs[i], osems[i])

    return emb_kernel(xT, lut2)


def kernel(x, lut):
    b, s = x.shape
    v, d = lut.shape
    xT = jnp.swapaxes(x, 0, 1).astype(jnp.int32)
    lutT = jnp.swapaxes(lut, 0, 1)
    ntail = v % BBLK
    tail = jnp.pad(lut[v - ntail:, :], ((0, 0), (0, PADW - d)))
    lut2 = _pack_table(lutT, tail, v)
    out = _embed(xT, lut2, s, b)
    return jnp.transpose(out, (2, 0, 1))
